# Initial kernel scaffold; baseline (speedup 1.0000x reference)
import jax
import jax.numpy as jnp
from jax.experimental import pallas as pl


def kernel(img):
    raise NotImplementedError("write your pallas kernel here")


# speedup vs baseline: 19.1480x; 19.1480x over previous
import functools

import jax
import jax.numpy as jnp
import numpy as np
from jax import lax
from jax.experimental import pallas as pl
from jax.experimental.pallas import tpu as pltpu
from jax.experimental.pallas import tpu_sc as plsc

ANGLE = 30.0

_NC = 2
_NS = 16
_NW = _NC * _NS

_H = 512
_W = 512
_BC = 384
_B = _H * _W
_B_PER_W = _B // _NW
_CHUNK = 128
_NCHUNK = _B_PER_W // _CHUNK


def _flat_src_index(w, h):
    xx, yy = jnp.meshgrid(jnp.arange(w), jnp.arange(h), indexing="ij")
    xx = xx.astype(jnp.float32)
    yy = yy.astype(jnp.float32)
    xm, ym = (w + 1) / 2.0, (h + 1) / 2.0
    inds = jnp.concatenate(
        [(xx - xm).reshape(-1, 1), (yy - ym).reshape(-1, 1)], axis=1)
    a = jnp.array([ANGLE * np.pi / 180.0], dtype=jnp.float32)
    c = jnp.cos(a)[0]
    s = jnp.sin(a)[0]
    R = jnp.array([[c, s], [-s, c]], dtype=jnp.float32)
    inds = jnp.round(R @ inds.T) + jnp.array([[xm], [ym]], dtype=jnp.float32)
    inds = jnp.where(inds < 0, 0.0, inds)
    row0 = jnp.where(inds[0, :] >= w, w - 1.0, inds[0, :])
    row1 = jnp.where(inds[1, :] >= h, h - 1.0, inds[1, :])
    iinds = jnp.stack([row0, row1], axis=0).astype(jnp.int32)
    return iinds[0, :] * h + iinds[1, :]


@functools.partial(
    pl.kernel,
    out_type=jax.ShapeDtypeStruct((_B, _BC), jnp.float32),
    mesh=plsc.VectorSubcoreMesh(
        core_axis_name="c", subcore_axis_name="s",
        num_cores=_NC, num_subcores=_NS),
    scratch_types=[
        pltpu.VMEM((_B_PER_W,), jnp.int32),
        pltpu.VMEM((_CHUNK, _BC), jnp.float32),
        pltpu.SemaphoreType.DMA,
    ],
)
def _sc_gather(table_hbm, idx_hbm, out_hbm, idx_v, rows_v, sem):
    wid = lax.axis_index("s") * _NC + lax.axis_index("c")
    base = wid * _B_PER_W
    pltpu.sync_copy(idx_hbm.at[pl.ds(base, _B_PER_W)], idx_v)

    def body(i, carry):
        pltpu.async_copy(
            table_hbm.at[idx_v.at[pl.ds(i * _CHUNK, _CHUNK)]], rows_v, sem
        ).wait()
        pltpu.sync_copy(rows_v, out_hbm.at[pl.ds(base + i * _CHUNK, _CHUNK)])
        return carry

    lax.fori_loop(0, _NCHUNK, body, 0)


def kernel(img):
    w, h = img.shape[-2], img.shape[-1]
    src = _flat_src_index(w, h)
    table = img.reshape(_BC, _B).T
    out_t = _sc_gather(table, src)
    return out_t.T.reshape(img.shape)
